# Initial kernel scaffold; baseline (speedup 1.0000x reference)
import jax
import jax.numpy as jnp
from jax.experimental import pallas as pl


def kernel(quantized_z, codebooks):
    raise NotImplementedError("write your pallas kernel here")


# speedup vs baseline: 1.9004x; 1.9004x over previous
import functools

import jax
import jax.numpy as jnp
from jax.experimental import pallas as pl

_N_BOOKS = 8
_K = 1024
_DIM = 32
_TB = 512


def _vq_kernel(z_ref, cb_ref, out_ref):
    z = z_ref[0]
    for i in range(_N_BOOKS):
        zi = z[i * _DIM:(i + 1) * _DIM, :]
        cb = cb_ref[i]
        y2 = jnp.sum(cb * cb, axis=1, keepdims=True)
        x2 = jnp.sum(zi * zi, axis=0, keepdims=True)
        dot = jax.lax.dot_general(
            cb, zi, (((1,), (0,)), ((), ())),
            preferred_element_type=jnp.float32)
        d2 = x2 + y2 - 2.0 * dot
        m = jnp.min(d2, axis=0, keepdims=True)
        ids = jax.lax.broadcasted_iota(jnp.int32, d2.shape, 0)
        idx = jnp.min(jnp.where(d2 <= m, ids, _K), axis=0)
        out_ref[0, i, :] = idx


@functools.partial(jax.jit, static_argnames=())
def kernel(quantized_z, codebooks):
    batch, total_feat, time = quantized_z.shape
    n_books, k, dim = codebooks.shape
    grid = (batch, time // _TB)
    return pl.pallas_call(
        _vq_kernel,
        grid=grid,
        in_specs=[
            pl.BlockSpec((1, total_feat, _TB), lambda b, t: (b, 0, t)),
            pl.BlockSpec((n_books, k, dim), lambda b, t: (0, 0, 0)),
        ],
        out_specs=pl.BlockSpec((1, n_books, _TB), lambda b, t: (b, 0, t)),
        out_shape=jax.ShapeDtypeStruct((batch, n_books, time), jnp.int32),
    )(quantized_z, codebooks)


# speedup vs baseline: 3.5852x; 1.8865x over previous
import functools

import jax
import jax.numpy as jnp
from jax.experimental import pallas as pl

_N_BOOKS = 8
_K = 1024
_DIM = 32
_TB = 512


def _vq_kernel(z_ref, cb_ref, out_ref):
    zs = z_ref[0] * -2.0
    for i in range(_N_BOOKS):
        zi = zs[i * _DIM:(i + 1) * _DIM, :]
        cb = cb_ref[i]
        y2 = jnp.sum(cb * cb, axis=1, keepdims=True)
        dot = jax.lax.dot_general(
            cb, zi, (((1,), (0,)), ((), ())),
            preferred_element_type=jnp.float32)
        scores = y2 + dot
        out_ref[0, i, :] = jnp.argmin(scores, axis=0).astype(jnp.int32)


@functools.partial(jax.jit, static_argnames=())
def kernel(quantized_z, codebooks):
    batch, total_feat, time = quantized_z.shape
    n_books, k, dim = codebooks.shape
    grid = (batch, time // _TB)
    return pl.pallas_call(
        _vq_kernel,
        grid=grid,
        in_specs=[
            pl.BlockSpec((1, total_feat, _TB), lambda b, t: (b, 0, t)),
            pl.BlockSpec((n_books, k, dim), lambda b, t: (0, 0, 0)),
        ],
        out_specs=pl.BlockSpec((1, n_books, _TB), lambda b, t: (b, 0, t)),
        out_shape=jax.ShapeDtypeStruct((batch, n_books, time), jnp.int32),
    )(quantized_z, codebooks)


# speedup vs baseline: 3.9498x; 1.1017x over previous
import functools

import jax
import jax.numpy as jnp
from jax.experimental import pallas as pl

_N_BOOKS = 8
_K = 1024
_DIM = 32
_TB = 2048


def _vq_kernel(z_ref, cb_ref, out_ref):
    zs = z_ref[0] * -2.0
    for i in range(_N_BOOKS):
        zi = zs[i * _DIM:(i + 1) * _DIM, :]
        cb = cb_ref[i]
        y2 = jnp.sum(cb * cb, axis=1, keepdims=True)
        dot = jax.lax.dot_general(
            cb, zi, (((1,), (0,)), ((), ())),
            preferred_element_type=jnp.float32)
        scores = y2 + dot
        out_ref[0, i, :] = jnp.argmin(scores, axis=0).astype(jnp.int32)


@functools.partial(jax.jit, static_argnames=())
def kernel(quantized_z, codebooks):
    batch, total_feat, time = quantized_z.shape
    n_books, k, dim = codebooks.shape
    grid = (batch, time // _TB)
    return pl.pallas_call(
        _vq_kernel,
        grid=grid,
        in_specs=[
            pl.BlockSpec((1, total_feat, _TB), lambda b, t: (b, 0, t)),
            pl.BlockSpec((n_books, k, dim), lambda b, t: (0, 0, 0)),
        ],
        out_specs=pl.BlockSpec((1, n_books, _TB), lambda b, t: (b, 0, t)),
        out_shape=jax.ShapeDtypeStruct((batch, n_books, time), jnp.int32),
    )(quantized_z, codebooks)


# speedup vs baseline: 4.4248x; 1.1203x over previous
import functools

import jax
import jax.numpy as jnp
from jax.experimental import pallas as pl

_N_BOOKS = 8
_K = 1024
_DIM = 32
_TB = 2048


def _vq_kernel(z_ref, cb_ref, out_ref):
    zs = z_ref[0] * -2.0
    ones = jnp.ones((1, zs.shape[1]), jnp.float32)
    for i in range(_N_BOOKS):
        zi = zs[i * _DIM:(i + 1) * _DIM, :]
        cb = cb_ref[i]
        y2 = jnp.sum(cb * cb, axis=1, keepdims=True)
        caug = jnp.concatenate([cb, y2], axis=1)
        zaug = jnp.concatenate([zi, ones], axis=0)
        scores = jax.lax.dot_general(
            caug, zaug, (((1,), (0,)), ((), ())),
            preferred_element_type=jnp.float32)
        out_ref[0, i, :] = jnp.argmin(scores, axis=0).astype(jnp.int32)


@functools.partial(jax.jit, static_argnames=())
def kernel(quantized_z, codebooks):
    batch, total_feat, time = quantized_z.shape
    n_books, k, dim = codebooks.shape
    grid = (batch, time // _TB)
    return pl.pallas_call(
        _vq_kernel,
        grid=grid,
        in_specs=[
            pl.BlockSpec((1, total_feat, _TB), lambda b, t: (b, 0, t)),
            pl.BlockSpec((n_books, k, dim), lambda b, t: (0, 0, 0)),
        ],
        out_specs=pl.BlockSpec((1, n_books, _TB), lambda b, t: (b, 0, t)),
        out_shape=jax.ShapeDtypeStruct((batch, n_books, time), jnp.int32),
    )(quantized_z, codebooks)
